# baseline (speedup 1.0000x reference)
import functools
import math

import jax
import jax.numpy as jnp
from jax import lax
from jax.experimental import pallas as pl
from jax.experimental.pallas import tpu as pltpu
from jax.experimental.pallas import tpu_sc as plsc

N = 10000
D = 128
E = 320000
H = 256
L = 4
EPS = 1e-7

LANES = 16
NT = 32
NPT = 320
NPAD = NT * NPT
CAP = 12288
EB = 128
NCH = 80
CH = E // NCH
NCHUNK = D // LANES

_MESH = plsc.VectorSubcoreMesh(core_axis_name="c", subcore_axis_name="s")
_SC_PARAMS = pltpu.CompilerParams(needs_layout_passes=False,
                                  use_tc_tiling_on_sc=False)


def _bucketize_body(src_hbm, dst_hbm, srcl_hbm, ldl_hbm, cnt_hbm,
                    dst_v, src_v, srcl_v, ldl_v, cnt_v):
    w = lax.axis_index("s") * 2 + lax.axis_index("c")
    lo = w * NPT
    lo_v = jnp.full((LANES,), lo, jnp.int32)
    hi_v = lo_v + NPT
    pad_ld = jnp.full((LANES,), NPT, jnp.int32)
    zero16 = jnp.zeros((LANES,), jnp.int32)

    def fill(i, carry):
        srcl_v[pl.ds(i * LANES, LANES)] = zero16
        ldl_v[pl.ds(i * LANES, LANES)] = pad_ld
        return carry
    lax.fori_loop(0, CAP // LANES, fill, 0)

    def chunk_body(ci, off):
        pltpu.sync_copy(dst_hbm.at[pl.ds(ci * CH, CH)], dst_v)
        pltpu.sync_copy(src_hbm.at[pl.ds(ci * CH, CH)], src_v)

        def scan_body(i, off):
            d16 = dst_v[pl.ds(i * LANES, LANES)]
            s16 = src_v[pl.ds(i * LANES, LANES)]
            m = (d16 >= lo_v) & (d16 < hi_v)
            cs = plsc.cumsum(m.astype(jnp.int32))
            idx = off + cs - 1
            plsc.store_scatter(srcl_v, [idx], s16, mask=m)
            plsc.store_scatter(ldl_v, [idx], d16 - lo_v, mask=m)
            return off + plsc.all_reduce_population_count(m)

        return lax.fori_loop(0, CH // LANES, scan_body, off)

    off = lax.fori_loop(0, NCH, chunk_body, zero16)
    cnt_v[...] = off
    pltpu.sync_copy(cnt_v, cnt_hbm.at[w])
    pltpu.sync_copy(srcl_v, srcl_hbm.at[w])
    pltpu.sync_copy(ldl_v, ldl_hbm.at[w])


_bucketize = functools.partial(
    pl.kernel,
    out_type=[
        jax.ShapeDtypeStruct((NT, CAP), jnp.int32),
        jax.ShapeDtypeStruct((NT, CAP), jnp.int32),
        jax.ShapeDtypeStruct((NT, LANES), jnp.int32),
    ],
    mesh=_MESH,
    scratch_types=[
        pltpu.VMEM((CH,), jnp.int32),
        pltpu.VMEM((CH,), jnp.int32),
        pltpu.VMEM((CAP,), jnp.int32),
        pltpu.VMEM((CAP,), jnp.int32),
        pltpu.VMEM((LANES,), jnp.int32),
    ],
    compiler_params=_SC_PARAMS,
)(_bucketize_body)


def _aggregate_body(numt_hbm, ext_hbm, srcl_hbm, ldl_hbm, cnt_hbm, out_hbm,
                    acc_v, nrows_a, nrows_b, erows_a, erows_b, idx_a, idx_b,
                    ldl_v, cnt_v, sem_a, sem_b):
    w = lax.axis_index("s") * 2 + lax.axis_index("c")
    zf = jnp.zeros((LANES,), jnp.float32)

    def zbody(i, carry):
        for q in range(2 * NCHUNK):
            acc_v[i, pl.ds(q * LANES, LANES)] = zf
        return carry
    lax.fori_loop(0, NPT + 1, zbody, 0)

    pltpu.sync_copy(cnt_hbm.at[w], cnt_v)
    pltpu.sync_copy(ldl_hbm.at[w], ldl_v)
    count = cnt_v[...][0]
    npair = (count + (2 * EB - 1)) // (2 * EB)

    def gather(b, nrows, erows, idx_v, sem):
        pltpu.sync_copy(srcl_hbm.at[w, pl.ds(b * EB, EB)], idx_v)
        pltpu.async_copy(numt_hbm.at[idx_v], nrows, sem)
        pltpu.async_copy(ext_hbm.at[idx_v], erows, sem)

    def gwait(nrows, erows, idx_v, sem):
        pltpu.make_async_copy(numt_hbm.at[idx_v], nrows, sem).wait()
        pltpu.make_async_copy(ext_hbm.at[idx_v], erows, sem).wait()

    def compute(nrows, erows, base):
        def group(g, carry2):
            ld16 = ldl_v[pl.ds(base + g * LANES, LANES)]
            for k in range(LANES):
                e = g * LANES + k
                ld = ld16[k]
                for cc in range(NCHUNK):
                    plsc.addupdate(acc_v.at[ld, pl.ds(cc * LANES, LANES)],
                                   nrows[e, pl.ds(cc * LANES, LANES)])
                    plsc.addupdate(
                        acc_v.at[ld, pl.ds(D + cc * LANES, LANES)],
                        erows[e, pl.ds(cc * LANES, LANES)])
            return carry2

        lax.fori_loop(0, EB // LANES, group, 0)

    def batch(b, carry):
        gather(b, nrows_a, erows_a, idx_a, sem_a)
        gwait(nrows_a, erows_a, idx_a, sem_a)
        compute(nrows_a, erows_a, b * EB)
        return carry

    lax.fori_loop(0, 2 * npair, batch, 0)

    pltpu.sync_copy(acc_v.at[pl.ds(0, NPT)],
                    out_hbm.at[pl.ds(w * NPT, NPT)])


_aggregate = functools.partial(
    pl.kernel,
    out_type=jax.ShapeDtypeStruct((NPAD, 2 * D), jnp.float32),
    mesh=_MESH,
    scratch_types=[
        pltpu.VMEM((NPT + 1, 2 * D), jnp.float32),
        pltpu.VMEM((EB, D), jnp.float32),
        pltpu.VMEM((EB, D), jnp.float32),
        pltpu.VMEM((EB, D), jnp.float32),
        pltpu.VMEM((EB, D), jnp.float32),
        pltpu.VMEM((EB,), jnp.int32),
        pltpu.VMEM((EB,), jnp.int32),
        pltpu.VMEM((CAP,), jnp.int32),
        pltpu.VMEM((LANES,), jnp.int32),
        pltpu.SemaphoreType.DMA,
        pltpu.SemaphoreType.DMA,
    ],
    compiler_params=_SC_PARAMS,
)(_aggregate_body)


BM = 400


def _colmax_update(m_ref, r):
    cur = jnp.max(r, axis=0, keepdims=True)

    @pl.when(pl.program_id(0) == 0)
    def _():
        m_ref[...] = cur

    @pl.when(pl.program_id(0) != 0)
    def _():
        m_ref[...] = jnp.maximum(m_ref[...], cur)


def _enc_body(x_ref, w_ref, b_ref, h_ref, m_ref):
    h = jnp.dot(x_ref[...], w_ref[...],
                preferred_element_type=jnp.float32) + b_ref[...]
    h_ref[...] = h
    _colmax_update(m_ref, jnp.maximum(h, 0.0))


def _encode(x, enc_W, enc_b):
    return pl.pallas_call(
        _enc_body,
        grid=(N // BM,),
        in_specs=[
            pl.BlockSpec((BM, D), lambda i: (i, 0)),
            pl.BlockSpec((D, D), lambda i: (0, 0)),
            pl.BlockSpec((1, D), lambda i: (0, 0)),
        ],
        out_specs=[
            pl.BlockSpec((BM, D), lambda i: (i, 0)),
            pl.BlockSpec((1, D), lambda i: (0, 0)),
        ],
        out_shape=[
            jax.ShapeDtypeStruct((N, D), jnp.float32),
            jax.ShapeDtypeStruct((1, D), jnp.float32),
        ],
    )(x, enc_W, enc_b.reshape(1, D))


def _table_body(base_ref, tv_ref, gm_ref, num_ref, ex_ref):
    msg = jnp.maximum(base_ref[...], 0.0) + EPS
    ex = jnp.exp(msg * tv_ref[...] - gm_ref[...])
    num_ref[...] = msg * ex
    ex_ref[...] = ex


def _table(base, tv, gm):
    nd_spec = pl.BlockSpec((BM, D), lambda i: (i, 0))
    row_d = pl.BlockSpec((1, D), lambda i: (0, 0))
    return pl.pallas_call(
        _table_body,
        grid=(N // BM,),
        in_specs=[nd_spec, row_d, row_d],
        out_specs=[nd_spec, nd_spec],
        out_shape=[
            jax.ShapeDtypeStruct((N, D), jnp.float32),
            jax.ShapeDtypeStruct((N, D), jnp.float32),
        ],
    )(base, tv, gm)


def _ln(z, g, b):
    mu = jnp.mean(z, axis=-1, keepdims=True)
    var = jnp.mean((z - mu) ** 2, axis=-1, keepdims=True)
    return (z - mu) * lax.rsqrt(var + 1e-5) * g + b


def _layer_body(num_ref, den_ref, base_ref, *rest, add_residual):
    if add_residual:
        hprev_ref = rest[0]
        rest = rest[1:]
    (w1_ref, b1_ref, g1_ref, bb1_ref, w2_ref, b2_ref, ng_ref, nb_ref,
     h_ref, r_ref, m_ref) = rest
    out = num_ref[...] / (den_ref[...] + 1e-16) + base_ref[...]
    z = jnp.dot(out, w1_ref[...],
                preferred_element_type=jnp.float32) + b1_ref[...]
    z = jnp.maximum(_ln(z, g1_ref[...], bb1_ref[...]), 0.0)
    hnew = jnp.dot(z, w2_ref[...],
                   preferred_element_type=jnp.float32) + b2_ref[...]
    if add_residual:
        hnew = hnew + hprev_ref[...]
    h_ref[...] = hnew
    r = jnp.maximum(_ln(hnew, ng_ref[...], nb_ref[...]), 0.0)
    r_ref[...] = r
    _colmax_update(m_ref, r)


def _layer(num, den, base, hprev, w1, b1, g1, bb1, w2, b2, ng, nbb):
    add_residual = hprev is not None
    nd_spec = pl.BlockSpec((BM, D), lambda i: (i, 0))
    row_d = pl.BlockSpec((1, D), lambda i: (0, 0))
    row_h = pl.BlockSpec((1, H), lambda i: (0, 0))
    in_specs = [nd_spec, nd_spec, nd_spec]
    args = [num, den, base]
    if add_residual:
        in_specs.append(nd_spec)
        args.append(hprev)
    in_specs += [
        pl.BlockSpec((D, H), lambda i: (0, 0)), row_h, row_h, row_h,
        pl.BlockSpec((H, D), lambda i: (0, 0)), row_d, row_d, row_d,
    ]
    args += [w1, b1.reshape(1, H), g1.reshape(1, H), bb1.reshape(1, H),
             w2, b2.reshape(1, D), ng.reshape(1, D), nbb.reshape(1, D)]
    return pl.pallas_call(
        functools.partial(_layer_body, add_residual=add_residual),
        grid=(N // BM,),
        in_specs=in_specs,
        out_specs=[nd_spec, nd_spec, row_d],
        out_shape=[
            jax.ShapeDtypeStruct((N, D), jnp.float32),
            jax.ShapeDtypeStruct((N, D), jnp.float32),
            jax.ShapeDtypeStruct((1, D), jnp.float32),
        ],
    )(*args)


_BN_INV = 1.0 / math.sqrt(1.0 + 1e-5)


def _final_body(num_ref, den_ref, base_ref, w1_ref, b1_ref, g_ref, bb_ref,
                w2_ref, b2_ref, y_ref):
    out = num_ref[...] / (den_ref[...] + 1e-16) + base_ref[...]
    z = jnp.dot(out, w1_ref[...],
                preferred_element_type=jnp.float32) + b1_ref[...]
    z = jnp.maximum(z * _BN_INV * g_ref[...] + bb_ref[...], 0.0)
    y_ref[...] = jnp.dot(z, w2_ref[...],
                         preferred_element_type=jnp.float32) + b2_ref[...]


def _final(num, den, base, w1, b1, g, bb, w2, b2):
    nd_spec = pl.BlockSpec((BM, D), lambda i: (i, 0))
    row_d = pl.BlockSpec((1, D), lambda i: (0, 0))
    row_h = pl.BlockSpec((1, H), lambda i: (0, 0))
    return pl.pallas_call(
        _final_body,
        grid=(N // BM,),
        in_specs=[nd_spec, nd_spec, nd_spec,
                  pl.BlockSpec((D, H), lambda i: (0, 0)), row_h, row_h, row_h,
                  pl.BlockSpec((H, D), lambda i: (0, 0)), row_d],
        out_specs=nd_spec,
        out_shape=jax.ShapeDtypeStruct((N, D), jnp.float32),
    )(num, den, base, w1, b1.reshape(1, H), g.reshape(1, H), bb.reshape(1, H),
      w2, b2.reshape(1, D))


def _round_agg(base, m, ti, srcl, ldl, cnt):
    gmax = jnp.maximum(ti * (m + EPS), ti * EPS)
    tv = jnp.full((1, D), ti, jnp.float32)
    numt, ext = _table(base, tv, gmax)
    agg = _aggregate(numt, ext, srcl, ldl, cnt)
    return agg[:N, :D], agg[:N, D:]


def kernel(x, edge_index, enc_W, enc_b, t, W1, b1, ln_g, ln_b, W2, b2,
           norm_g, norm_b, fin_W1, fin_b1, fin_bn_g, fin_bn_b, fin_W2,
           fin_b2):
    src = edge_index[0].astype(jnp.int32)
    dst = edge_index[1].astype(jnp.int32)
    srcl, ldl, cnt = _bucketize(src, dst)
    h0, m0 = _encode(x, enc_W, enc_b)
    h = None
    base = h0
    m = m0
    for i in range(L):
        num, den = _round_agg(base, m, t[i], srcl, ldl, cnt)
        j = i + 1 if i + 1 < L else 0
        h, base, m = _layer(num, den, base, h if i > 0 else None,
                            W1[i], b1[i], ln_g[i], ln_b[i], W2[i], b2[i],
                            norm_g[j], norm_b[j])
    num, den = _round_agg(base, m, jnp.float32(1.0), srcl, ldl, cnt)
    return _final(num, den, base, fin_W1, fin_b1, fin_bn_g, fin_bn_b,
                  fin_W2, fin_b2)


# speedup vs baseline: 1.1881x; 1.1881x over previous
import functools
import math

import jax
import jax.numpy as jnp
from jax import lax
from jax.experimental import pallas as pl
from jax.experimental.pallas import tpu as pltpu
from jax.experimental.pallas import tpu_sc as plsc

N = 10000
D = 128
E = 320000
H = 256
L = 4
EPS = 1e-7

LANES = 16
NT = 32
NPT = 320
NPAD = NT * NPT
CAP = 12288
EB = 64
NCH = 80
CH = E // NCH
NCHUNK = D // LANES

_MESH = plsc.VectorSubcoreMesh(core_axis_name="c", subcore_axis_name="s")
_SC_PARAMS = pltpu.CompilerParams(needs_layout_passes=False,
                                  use_tc_tiling_on_sc=False)


def _bucketize_body(src_hbm, dst_hbm, srcl_hbm, ldl_hbm, cnt_hbm,
                    dst_v, src_v, srcl_v, ldl_v, cnt_v):
    w = lax.axis_index("s") * 2 + lax.axis_index("c")
    lo = w * NPT
    lo_v = jnp.full((LANES,), lo, jnp.int32)
    hi_v = lo_v + NPT
    pad_ld = jnp.full((LANES,), NPT, jnp.int32)
    zero16 = jnp.zeros((LANES,), jnp.int32)

    def fill(i, carry):
        srcl_v[pl.ds(i * LANES, LANES)] = zero16
        ldl_v[pl.ds(i * LANES, LANES)] = pad_ld
        return carry
    lax.fori_loop(0, CAP // LANES, fill, 0)

    def chunk_body(ci, off):
        pltpu.sync_copy(dst_hbm.at[pl.ds(ci * CH, CH)], dst_v)
        pltpu.sync_copy(src_hbm.at[pl.ds(ci * CH, CH)], src_v)

        def scan_body(i, off):
            d16 = dst_v[pl.ds(i * LANES, LANES)]
            s16 = src_v[pl.ds(i * LANES, LANES)]
            m = (d16 >= lo_v) & (d16 < hi_v)
            cs = plsc.cumsum(m.astype(jnp.int32))
            idx = off + cs - 1
            plsc.store_scatter(srcl_v, [idx], s16, mask=m)
            plsc.store_scatter(ldl_v, [idx], d16 - lo_v, mask=m)
            return off + plsc.all_reduce_population_count(m)

        return lax.fori_loop(0, CH // LANES, scan_body, off)

    off = lax.fori_loop(0, NCH, chunk_body, zero16)
    cnt_v[...] = off
    pltpu.sync_copy(cnt_v, cnt_hbm.at[w])
    pltpu.sync_copy(srcl_v, srcl_hbm.at[w])
    pltpu.sync_copy(ldl_v, ldl_hbm.at[w])


_bucketize = functools.partial(
    pl.kernel,
    out_type=[
        jax.ShapeDtypeStruct((NT, CAP), jnp.int32),
        jax.ShapeDtypeStruct((NT, CAP), jnp.int32),
        jax.ShapeDtypeStruct((NT, LANES), jnp.int32),
    ],
    mesh=_MESH,
    scratch_types=[
        pltpu.VMEM((CH,), jnp.int32),
        pltpu.VMEM((CH,), jnp.int32),
        pltpu.VMEM((CAP,), jnp.int32),
        pltpu.VMEM((CAP,), jnp.int32),
        pltpu.VMEM((LANES,), jnp.int32),
    ],
    compiler_params=_SC_PARAMS,
)(_bucketize_body)


def _aggregate_body(numt_hbm, ext_hbm, srcl_hbm, ldl_hbm, cnt_hbm, out_hbm,
                    acc_v, nrows_a, nrows_b, erows_a, erows_b, idx_a, idx_b,
                    ldl_v, cnt_v, sem_a, sem_b):
    w = lax.axis_index("s") * 2 + lax.axis_index("c")
    zf = jnp.zeros((LANES,), jnp.float32)

    def zbody(i, carry):
        for q in range(2 * NCHUNK):
            acc_v[i, pl.ds(q * LANES, LANES)] = zf
        return carry
    lax.fori_loop(0, NPT + 1, zbody, 0)

    pltpu.sync_copy(cnt_hbm.at[w], cnt_v)
    pltpu.sync_copy(ldl_hbm.at[w], ldl_v)
    count = cnt_v[...][0]
    npair = (count + (2 * EB - 1)) // (2 * EB)

    def gather(b, nrows, erows, idx_v, sem):
        pltpu.sync_copy(srcl_hbm.at[w, pl.ds(b * EB, EB)], idx_v)
        pltpu.async_copy(numt_hbm.at[idx_v], nrows, sem)
        pltpu.async_copy(ext_hbm.at[idx_v], erows, sem)

    def gwait(nrows, erows, idx_v, sem):
        pltpu.make_async_copy(numt_hbm.at[idx_v], nrows, sem).wait()
        pltpu.make_async_copy(ext_hbm.at[idx_v], erows, sem).wait()

    def compute(nrows, erows, base):
        def group(g, carry2):
            ld16 = ldl_v[pl.ds(base + g * LANES, LANES)]
            for k in range(LANES):
                e = g * LANES + k
                ld = ld16[k]
                for cc in range(NCHUNK):
                    plsc.addupdate(acc_v.at[ld, pl.ds(cc * LANES, LANES)],
                                   nrows[e, pl.ds(cc * LANES, LANES)])
                    plsc.addupdate(
                        acc_v.at[ld, pl.ds(D + cc * LANES, LANES)],
                        erows[e, pl.ds(cc * LANES, LANES)])
            return carry2

        lax.fori_loop(0, EB // LANES, group, 0)

    @pl.when(npair > 0)
    def _():
        gather(0, nrows_a, erows_a, idx_a, sem_a)

        def pair(p, carry):
            gather(2 * p + 1, nrows_b, erows_b, idx_b, sem_b)
            gwait(nrows_a, erows_a, idx_a, sem_a)
            compute(nrows_a, erows_a, (2 * p) * EB)

            @pl.when(p + 1 < npair)
            def _():
                gather(2 * p + 2, nrows_a, erows_a, idx_a, sem_a)

            gwait(nrows_b, erows_b, idx_b, sem_b)
            compute(nrows_b, erows_b, (2 * p + 1) * EB)
            return carry

        lax.fori_loop(0, npair, pair, 0)

    pltpu.sync_copy(acc_v.at[pl.ds(0, NPT)],
                    out_hbm.at[pl.ds(w * NPT, NPT)])


_aggregate = functools.partial(
    pl.kernel,
    out_type=jax.ShapeDtypeStruct((NPAD, 2 * D), jnp.float32),
    mesh=_MESH,
    scratch_types=[
        pltpu.VMEM((NPT + 1, 2 * D), jnp.float32),
        pltpu.VMEM((EB, D), jnp.float32),
        pltpu.VMEM((EB, D), jnp.float32),
        pltpu.VMEM((EB, D), jnp.float32),
        pltpu.VMEM((EB, D), jnp.float32),
        pltpu.VMEM((EB,), jnp.int32),
        pltpu.VMEM((EB,), jnp.int32),
        pltpu.VMEM((CAP,), jnp.int32),
        pltpu.VMEM((LANES,), jnp.int32),
        pltpu.SemaphoreType.DMA,
        pltpu.SemaphoreType.DMA,
    ],
    compiler_params=_SC_PARAMS,
)(_aggregate_body)


BM = 400


def _colmax_update(m_ref, r):
    cur = jnp.max(r, axis=0, keepdims=True)

    @pl.when(pl.program_id(0) == 0)
    def _():
        m_ref[...] = cur

    @pl.when(pl.program_id(0) != 0)
    def _():
        m_ref[...] = jnp.maximum(m_ref[...], cur)


def _enc_body(x_ref, w_ref, b_ref, h_ref, m_ref):
    h = jnp.dot(x_ref[...], w_ref[...],
                preferred_element_type=jnp.float32) + b_ref[...]
    h_ref[...] = h
    _colmax_update(m_ref, jnp.maximum(h, 0.0))


def _encode(x, enc_W, enc_b):
    return pl.pallas_call(
        _enc_body,
        grid=(N // BM,),
        in_specs=[
            pl.BlockSpec((BM, D), lambda i: (i, 0)),
            pl.BlockSpec((D, D), lambda i: (0, 0)),
            pl.BlockSpec((1, D), lambda i: (0, 0)),
        ],
        out_specs=[
            pl.BlockSpec((BM, D), lambda i: (i, 0)),
            pl.BlockSpec((1, D), lambda i: (0, 0)),
        ],
        out_shape=[
            jax.ShapeDtypeStruct((N, D), jnp.float32),
            jax.ShapeDtypeStruct((1, D), jnp.float32),
        ],
    )(x, enc_W, enc_b.reshape(1, D))


def _table_body(base_ref, tv_ref, gm_ref, num_ref, ex_ref):
    msg = jnp.maximum(base_ref[...], 0.0) + EPS
    ex = jnp.exp(msg * tv_ref[...] - gm_ref[...])
    num_ref[...] = msg * ex
    ex_ref[...] = ex


def _table(base, tv, gm):
    nd_spec = pl.BlockSpec((BM, D), lambda i: (i, 0))
    row_d = pl.BlockSpec((1, D), lambda i: (0, 0))
    return pl.pallas_call(
        _table_body,
        grid=(N // BM,),
        in_specs=[nd_spec, row_d, row_d],
        out_specs=[nd_spec, nd_spec],
        out_shape=[
            jax.ShapeDtypeStruct((N, D), jnp.float32),
            jax.ShapeDtypeStruct((N, D), jnp.float32),
        ],
    )(base, tv, gm)


def _ln(z, g, b):
    mu = jnp.mean(z, axis=-1, keepdims=True)
    var = jnp.mean((z - mu) ** 2, axis=-1, keepdims=True)
    return (z - mu) * lax.rsqrt(var + 1e-5) * g + b


def _layer_body(num_ref, den_ref, base_ref, *rest, add_residual):
    if add_residual:
        hprev_ref = rest[0]
        rest = rest[1:]
    (w1_ref, b1_ref, g1_ref, bb1_ref, w2_ref, b2_ref, ng_ref, nb_ref,
     h_ref, r_ref, m_ref) = rest
    out = num_ref[...] / (den_ref[...] + 1e-16) + base_ref[...]
    z = jnp.dot(out, w1_ref[...],
                preferred_element_type=jnp.float32) + b1_ref[...]
    z = jnp.maximum(_ln(z, g1_ref[...], bb1_ref[...]), 0.0)
    hnew = jnp.dot(z, w2_ref[...],
                   preferred_element_type=jnp.float32) + b2_ref[...]
    if add_residual:
        hnew = hnew + hprev_ref[...]
    h_ref[...] = hnew
    r = jnp.maximum(_ln(hnew, ng_ref[...], nb_ref[...]), 0.0)
    r_ref[...] = r
    _colmax_update(m_ref, r)


def _layer(num, den, base, hprev, w1, b1, g1, bb1, w2, b2, ng, nbb):
    add_residual = hprev is not None
    nd_spec = pl.BlockSpec((BM, D), lambda i: (i, 0))
    row_d = pl.BlockSpec((1, D), lambda i: (0, 0))
    row_h = pl.BlockSpec((1, H), lambda i: (0, 0))
    in_specs = [nd_spec, nd_spec, nd_spec]
    args = [num, den, base]
    if add_residual:
        in_specs.append(nd_spec)
        args.append(hprev)
    in_specs += [
        pl.BlockSpec((D, H), lambda i: (0, 0)), row_h, row_h, row_h,
        pl.BlockSpec((H, D), lambda i: (0, 0)), row_d, row_d, row_d,
    ]
    args += [w1, b1.reshape(1, H), g1.reshape(1, H), bb1.reshape(1, H),
             w2, b2.reshape(1, D), ng.reshape(1, D), nbb.reshape(1, D)]
    return pl.pallas_call(
        functools.partial(_layer_body, add_residual=add_residual),
        grid=(N // BM,),
        in_specs=in_specs,
        out_specs=[nd_spec, nd_spec, row_d],
        out_shape=[
            jax.ShapeDtypeStruct((N, D), jnp.float32),
            jax.ShapeDtypeStruct((N, D), jnp.float32),
            jax.ShapeDtypeStruct((1, D), jnp.float32),
        ],
    )(*args)


_BN_INV = 1.0 / math.sqrt(1.0 + 1e-5)


def _final_body(num_ref, den_ref, base_ref, w1_ref, b1_ref, g_ref, bb_ref,
                w2_ref, b2_ref, y_ref):
    out = num_ref[...] / (den_ref[...] + 1e-16) + base_ref[...]
    z = jnp.dot(out, w1_ref[...],
                preferred_element_type=jnp.float32) + b1_ref[...]
    z = jnp.maximum(z * _BN_INV * g_ref[...] + bb_ref[...], 0.0)
    y_ref[...] = jnp.dot(z, w2_ref[...],
                         preferred_element_type=jnp.float32) + b2_ref[...]


def _final(num, den, base, w1, b1, g, bb, w2, b2):
    nd_spec = pl.BlockSpec((BM, D), lambda i: (i, 0))
    row_d = pl.BlockSpec((1, D), lambda i: (0, 0))
    row_h = pl.BlockSpec((1, H), lambda i: (0, 0))
    return pl.pallas_call(
        _final_body,
        grid=(N // BM,),
        in_specs=[nd_spec, nd_spec, nd_spec,
                  pl.BlockSpec((D, H), lambda i: (0, 0)), row_h, row_h, row_h,
                  pl.BlockSpec((H, D), lambda i: (0, 0)), row_d],
        out_specs=nd_spec,
        out_shape=jax.ShapeDtypeStruct((N, D), jnp.float32),
    )(num, den, base, w1, b1.reshape(1, H), g.reshape(1, H), bb.reshape(1, H),
      w2, b2.reshape(1, D))


def _round_agg(base, m, ti, srcl, ldl, cnt):
    gmax = jnp.maximum(ti * (m + EPS), ti * EPS)
    tv = jnp.full((1, D), ti, jnp.float32)
    numt, ext = _table(base, tv, gmax)
    agg = _aggregate(numt, ext, srcl, ldl, cnt)
    return agg[:N, :D], agg[:N, D:]


def kernel(x, edge_index, enc_W, enc_b, t, W1, b1, ln_g, ln_b, W2, b2,
           norm_g, norm_b, fin_W1, fin_b1, fin_bn_g, fin_bn_b, fin_W2,
           fin_b2):
    src = edge_index[0].astype(jnp.int32)
    dst = edge_index[1].astype(jnp.int32)
    srcl, ldl, cnt = _bucketize(src, dst)
    h0, m0 = _encode(x, enc_W, enc_b)
    h = None
    base = h0
    m = m0
    for i in range(L):
        num, den = _round_agg(base, m, t[i], srcl, ldl, cnt)
        j = i + 1 if i + 1 < L else 0
        h, base, m = _layer(num, den, base, h if i > 0 else None,
                            W1[i], b1[i], ln_g[i], ln_b[i], W2[i], b2[i],
                            norm_g[j], norm_b[j])
    num, den = _round_agg(base, m, jnp.float32(1.0), srcl, ldl, cnt)
    return _final(num, den, base, fin_W1, fin_b1, fin_bn_g, fin_bn_b,
                  fin_W2, fin_b2)
